# baseline (speedup 1.0000x reference)
import functools

import jax
import jax.numpy as jnp
from jax import lax
from jax.experimental import pallas as pl
from jax.experimental.pallas import tpu as pltpu
from jax.experimental.pallas import tpu_sc as plsc

N = 10000
E = 320000
D = 128

NP = 10240
NC = 2
NS = 16
CHUNK = 128
KT = 160
KA = 80
KW = 80
EP = NS * KT * CHUNK
RPT = NP // NS

BLK = 512
NBLK = NP // BLK

_mesh = plsc.VectorSubcoreMesh(core_axis_name="c", subcore_axis_name="s")


@functools.partial(
    pl.kernel,
    out_type=jax.ShapeDtypeStruct((NC, NP), jnp.float32),
    mesh=_mesh,
    scratch_types=[
        pltpu.VMEM((KW, CHUNK), jnp.int32),
        pltpu.VMEM((CHUNK,), jnp.float32),
        pltpu.VMEM((RPT,), jnp.float32),
        pltpu.VMEM_SHARED((NP,), jnp.float32),
    ],
)
def _sc_degree(col_hbm, deg_hbm, colbuf, ones_v, zero_v, acc):
    c = lax.axis_index("c")
    s = lax.axis_index("s")

    ofs = jnp.where(c == 0, 0, KA)
    cnt = jnp.where(c == 0, KA, KT - KA)
    pltpu.sync_copy(col_hbm.at[s, pl.ds(ofs, KW)], colbuf)

    def fill(j, carry):
        ones_v[pl.ds(j * 16, 16)] = jnp.full((16,), 1.0, jnp.float32)
        return carry
    lax.fori_loop(0, CHUNK // 16, fill, 0)

    def zfill(j, carry):
        zero_v[pl.ds(j * 16, 16)] = jnp.zeros((16,), jnp.float32)
        return carry
    lax.fori_loop(0, RPT // 16, zfill, 0)

    pltpu.sync_copy(zero_v, acc.at[pl.ds(s * RPT, RPT)])
    plsc.subcore_barrier()

    def step(j, carry):
        pltpu.sync_copy(ones_v, acc.at[colbuf.at[j]], add=True)
        return carry
    lax.fori_loop(0, cnt, step, 0)

    plsc.subcore_barrier()
    pltpu.sync_copy(acc.at[pl.ds(s * RPT, RPT)],
                    deg_hbm.at[c, pl.ds(s * RPT, RPT)])


@functools.partial(
    pl.kernel,
    out_type=jax.ShapeDtypeStruct((NC, NP, D), jnp.float32),
    mesh=_mesh,
    scratch_types=[
        pltpu.VMEM((KW, CHUNK), jnp.int32),
        pltpu.VMEM((KW, CHUNK), jnp.int32),
        pltpu.VMEM((CHUNK, D), jnp.float32),
        pltpu.VMEM_SHARED((NP, D), jnp.float32),
        pltpu.SemaphoreType.DMA,
    ],
)
def _sc_scatter(hp_hbm, row_hbm, col_hbm, z_hbm, out_hbm,
                rowbuf, colbuf, msg0, acc, sem0):
    c = lax.axis_index("c")
    s = lax.axis_index("s")

    ofs = jnp.where(c == 0, 0, KA)
    cnt = jnp.where(c == 0, KA, KT - KA)
    pltpu.sync_copy(row_hbm.at[s, pl.ds(ofs, KW)], rowbuf)
    pltpu.sync_copy(col_hbm.at[s, pl.ds(ofs, KW)], colbuf)
    pltpu.sync_copy(z_hbm.at[pl.ds(s * RPT, RPT)], acc.at[pl.ds(s * RPT, RPT)])
    plsc.subcore_barrier()

    def step(j, carry):
        pltpu.async_copy(hp_hbm.at[rowbuf.at[j]], msg0, sem0).wait()
        pltpu.sync_copy(msg0, acc.at[colbuf.at[j]], add=True)
        return carry
    lax.fori_loop(0, cnt, step, 0)

    plsc.subcore_barrier()
    pltpu.sync_copy(acc.at[pl.ds(s * RPT, RPT)],
                    out_hbm.at[c, pl.ds(s * RPT, RPT)])


def _mm_body(x_ref, w_ref, out_ref):
    out_ref[...] = jnp.dot(x_ref[...], w_ref[...],
                           preferred_element_type=jnp.float32)


def _tc_mm(x, w):
    return pl.pallas_call(
        _mm_body,
        grid=(NBLK,),
        in_specs=[
            pl.BlockSpec((BLK, D), lambda i: (i, 0)),
            pl.BlockSpec((D, D), lambda i: (0, 0)),
        ],
        out_specs=pl.BlockSpec((BLK, D), lambda i: (i, 0)),
        out_shape=jax.ShapeDtypeStruct((NP, D), jnp.float32),
    )(x, w)


def _scale_body(h_ref, d0_ref, d1_ref, hp_ref, dinv_ref):
    dinv = lax.rsqrt(d0_ref[...] + d1_ref[...] + 1.0)
    dinv_ref[...] = dinv
    hp_ref[...] = h_ref[...] * dinv


def _tc_scale(h, d0, d1):
    return pl.pallas_call(
        _scale_body,
        grid=(NBLK,),
        in_specs=[
            pl.BlockSpec((BLK, D), lambda i: (i, 0)),
            pl.BlockSpec((BLK, 1), lambda i: (i, 0)),
            pl.BlockSpec((BLK, 1), lambda i: (i, 0)),
        ],
        out_specs=[
            pl.BlockSpec((BLK, D), lambda i: (i, 0)),
            pl.BlockSpec((BLK, 1), lambda i: (i, 0)),
        ],
        out_shape=[
            jax.ShapeDtypeStruct((NP, D), jnp.float32),
            jax.ShapeDtypeStruct((NP, 1), jnp.float32),
        ],
    )(h, d0, d1)


def _layer_body(s0_ref, s1_ref, hp_ref, dinv_ref, b_ref, w_ref, out_ref):
    a = s0_ref[...] + s1_ref[...] + hp_ref[...]
    a = jnp.maximum(a * dinv_ref[...] + b_ref[...], 0.0)
    h = jnp.dot(a, w_ref[...], preferred_element_type=jnp.float32)
    out_ref[...] = h * dinv_ref[...]


def _tc_layer(s0, s1, hp, dinv_col, b, w):
    return pl.pallas_call(
        _layer_body,
        grid=(NBLK,),
        in_specs=[
            pl.BlockSpec((BLK, D), lambda i: (i, 0)),
            pl.BlockSpec((BLK, D), lambda i: (i, 0)),
            pl.BlockSpec((BLK, D), lambda i: (i, 0)),
            pl.BlockSpec((BLK, 1), lambda i: (i, 0)),
            pl.BlockSpec((1, D), lambda i: (0, 0)),
            pl.BlockSpec((D, D), lambda i: (0, 0)),
        ],
        out_specs=pl.BlockSpec((BLK, D), lambda i: (i, 0)),
        out_shape=jax.ShapeDtypeStruct((NP, D), jnp.float32),
    )(s0, s1, hp, dinv_col, b, w)


def _final_body(s0_ref, s1_ref, hp_ref, dinv_ref, b_ref, out_ref):
    a = s0_ref[...] + s1_ref[...] + hp_ref[...]
    out_ref[...] = jnp.maximum(a * dinv_ref[...] + b_ref[...], 0.0)


def _tc_final(s0, s1, hp, dinv_col, b):
    return pl.pallas_call(
        _final_body,
        grid=(NBLK,),
        in_specs=[
            pl.BlockSpec((BLK, D), lambda i: (i, 0)),
            pl.BlockSpec((BLK, D), lambda i: (i, 0)),
            pl.BlockSpec((BLK, D), lambda i: (i, 0)),
            pl.BlockSpec((BLK, 1), lambda i: (i, 0)),
            pl.BlockSpec((1, D), lambda i: (0, 0)),
        ],
        out_specs=pl.BlockSpec((BLK, D), lambda i: (i, 0)),
        out_shape=jax.ShapeDtypeStruct((NP, D), jnp.float32),
    )(s0, s1, hp, dinv_col, b)


def kernel(x, edge_index, W1, b1, W2, b2):
    pad_e = EP - E
    row = jnp.concatenate(
        [edge_index[0], N + 16 + (jnp.arange(pad_e, dtype=jnp.int32) % 16)]
    ).reshape(NS, KT, CHUNK)
    col = jnp.concatenate(
        [edge_index[1], N + (jnp.arange(pad_e, dtype=jnp.int32) % 16)]
    ).reshape(NS, KT, CHUNK)

    x_pad = jnp.concatenate([x, jnp.zeros((NP - N, D), x.dtype)], axis=0)
    zeros2 = jnp.zeros((NP, D), jnp.float32)

    deg2 = _sc_degree(col)
    h1r = _tc_mm(x_pad, W1)
    h1, dinv_col = _tc_scale(h1r, deg2[0].reshape(NP, 1),
                             deg2[1].reshape(NP, 1))
    s1 = _sc_scatter(h1, row, col, zeros2)
    h2 = _tc_layer(s1[0], s1[1], h1, dinv_col, b1.reshape(1, D), W2)
    s2 = _sc_scatter(h2, row, col, zeros2)
    out = _tc_final(s2[0], s2[1], h2, dinv_col, b2.reshape(1, D))
    return out[:N]


# speedup vs baseline: 1.0050x; 1.0050x over previous
import functools

import jax
import jax.numpy as jnp
from jax import lax
from jax.experimental import pallas as pl
from jax.experimental.pallas import tpu as pltpu
from jax.experimental.pallas import tpu_sc as plsc

N = 10000
E = 320000
D = 128

NP = 10240
NC = 2
NS = 16
CHUNK = 128
KT = 160
KA = 80
KW = 80
EP = NS * KT * CHUNK
RPT = NP // NS

BLK = 512
NBLK = NP // BLK

_mesh = plsc.VectorSubcoreMesh(core_axis_name="c", subcore_axis_name="s")


@functools.partial(
    pl.kernel,
    out_type=jax.ShapeDtypeStruct((NC, NP), jnp.float32),
    mesh=_mesh,
    scratch_types=[
        pltpu.VMEM((KW, CHUNK), jnp.int32),
        pltpu.VMEM((CHUNK,), jnp.float32),
        pltpu.VMEM((RPT,), jnp.float32),
        pltpu.VMEM_SHARED((NP,), jnp.float32),
    ],
)
def _sc_degree(col_hbm, deg_hbm, colbuf, ones_v, zero_v, acc):
    c = lax.axis_index("c")
    s = lax.axis_index("s")

    ofs = jnp.where(c == 0, 0, KA)
    cnt = jnp.where(c == 0, KA, KT - KA)
    pltpu.sync_copy(col_hbm.at[s, pl.ds(ofs, KW)], colbuf)

    def fill(j, carry):
        ones_v[pl.ds(j * 16, 16)] = jnp.full((16,), 1.0, jnp.float32)
        return carry
    lax.fori_loop(0, CHUNK // 16, fill, 0)

    def zfill(j, carry):
        zero_v[pl.ds(j * 16, 16)] = jnp.zeros((16,), jnp.float32)
        return carry
    lax.fori_loop(0, RPT // 16, zfill, 0)

    pltpu.sync_copy(zero_v, acc.at[pl.ds(s * RPT, RPT)])
    plsc.subcore_barrier()

    def step(j, carry):
        pltpu.sync_copy(ones_v, acc.at[colbuf.at[j]], add=True)
        return carry
    lax.fori_loop(0, cnt, step, 0)

    plsc.subcore_barrier()
    pltpu.sync_copy(acc.at[pl.ds(s * RPT, RPT)],
                    deg_hbm.at[c, pl.ds(s * RPT, RPT)])


@functools.partial(
    pl.kernel,
    out_type=jax.ShapeDtypeStruct((NC, NP, D), jnp.float32),
    mesh=_mesh,
    scratch_types=[
        pltpu.VMEM((KW, CHUNK), jnp.int32),
        pltpu.VMEM((KW, CHUNK), jnp.int32),
        pltpu.VMEM((CHUNK, D), jnp.float32),
        pltpu.VMEM_SHARED((NP, D), jnp.float32),
        pltpu.SemaphoreType.DMA,
    ],
)
def _sc_scatter(hp_hbm, row_hbm, col_hbm, z_hbm, out_hbm,
                rowbuf, colbuf, msg0, acc, sem0):
    c = lax.axis_index("c")
    s = lax.axis_index("s")

    ofs = jnp.where(c == 0, 0, KA)
    cnt = jnp.where(c == 0, KA, KT - KA)
    pltpu.sync_copy(row_hbm.at[s, pl.ds(ofs, KW)], rowbuf)
    pltpu.sync_copy(col_hbm.at[s, pl.ds(ofs, KW)], colbuf)
    pltpu.sync_copy(z_hbm.at[pl.ds(s * RPT, RPT)], acc.at[pl.ds(s * RPT, RPT)])
    plsc.subcore_barrier()

    def step(j, carry):
        pltpu.async_copy(hp_hbm.at[rowbuf.at[j]], msg0, sem0).wait()
        pltpu.sync_copy(msg0, acc.at[colbuf.at[j]], add=True)
        return carry
    lax.fori_loop(0, cnt, step, 0)

    plsc.subcore_barrier()
    pltpu.sync_copy(acc.at[pl.ds(s * RPT, RPT)],
                    out_hbm.at[c, pl.ds(s * RPT, RPT)])


def _mm_body(x_ref, w_ref, out_ref):
    out_ref[...] = jnp.dot(x_ref[...], w_ref[...],
                           preferred_element_type=jnp.float32)


def _tc_mm(x, w):
    return pl.pallas_call(
        _mm_body,
        grid=(NBLK,),
        in_specs=[
            pl.BlockSpec((BLK, D), lambda i: (i, 0)),
            pl.BlockSpec((D, D), lambda i: (0, 0)),
        ],
        out_specs=pl.BlockSpec((BLK, D), lambda i: (i, 0)),
        out_shape=jax.ShapeDtypeStruct((NP, D), jnp.float32),
    )(x, w)


def _scale_body(h_ref, d0_ref, d1_ref, hp_ref, dinv_ref):
    dinv = lax.rsqrt(d0_ref[...] + d1_ref[...] + 1.0)
    dinv_ref[...] = dinv
    hp_ref[...] = h_ref[...] * dinv


def _tc_scale(h, d0, d1):
    return pl.pallas_call(
        _scale_body,
        grid=(NBLK,),
        in_specs=[
            pl.BlockSpec((BLK, D), lambda i: (i, 0)),
            pl.BlockSpec((BLK, 1), lambda i: (i, 0)),
            pl.BlockSpec((BLK, 1), lambda i: (i, 0)),
        ],
        out_specs=[
            pl.BlockSpec((BLK, D), lambda i: (i, 0)),
            pl.BlockSpec((BLK, 1), lambda i: (i, 0)),
        ],
        out_shape=[
            jax.ShapeDtypeStruct((NP, D), jnp.float32),
            jax.ShapeDtypeStruct((NP, 1), jnp.float32),
        ],
    )(h, d0, d1)


def _layer_body(s0_ref, s1_ref, hp_ref, dinv_ref, b_ref, w_ref, out_ref):
    a = s0_ref[...] + s1_ref[...] + hp_ref[...]
    a = jnp.maximum(a * dinv_ref[...] + b_ref[...], 0.0)
    h = jnp.dot(a, w_ref[...], preferred_element_type=jnp.float32)
    out_ref[...] = h * dinv_ref[...]


def _tc_layer(s0, s1, hp, dinv_col, b, w):
    return pl.pallas_call(
        _layer_body,
        grid=(NBLK,),
        in_specs=[
            pl.BlockSpec((BLK, D), lambda i: (i, 0)),
            pl.BlockSpec((BLK, D), lambda i: (i, 0)),
            pl.BlockSpec((BLK, D), lambda i: (i, 0)),
            pl.BlockSpec((BLK, 1), lambda i: (i, 0)),
            pl.BlockSpec((1, D), lambda i: (0, 0)),
            pl.BlockSpec((D, D), lambda i: (0, 0)),
        ],
        out_specs=pl.BlockSpec((BLK, D), lambda i: (i, 0)),
        out_shape=jax.ShapeDtypeStruct((NP, D), jnp.float32),
    )(s0, s1, hp, dinv_col, b, w)


def _final_body(s0_ref, s1_ref, hp_ref, dinv_ref, b_ref, out_ref):
    a = s0_ref[...] + s1_ref[...] + hp_ref[...]
    out_ref[...] = jnp.maximum(a * dinv_ref[...] + b_ref[...], 0.0)


def _tc_final(s0, s1, hp, dinv_col, b):
    return pl.pallas_call(
        _final_body,
        grid=(N // 400,),
        in_specs=[
            pl.BlockSpec((400, D), lambda i: (i, 0)),
            pl.BlockSpec((400, D), lambda i: (i, 0)),
            pl.BlockSpec((400, D), lambda i: (i, 0)),
            pl.BlockSpec((400, 1), lambda i: (i, 0)),
            pl.BlockSpec((1, D), lambda i: (0, 0)),
        ],
        out_specs=pl.BlockSpec((400, D), lambda i: (i, 0)),
        out_shape=jax.ShapeDtypeStruct((N, D), jnp.float32),
    )(s0, s1, hp, dinv_col, b)


def kernel(x, edge_index, W1, b1, W2, b2):
    pad_e = EP - E
    row = jnp.concatenate(
        [edge_index[0], N + 16 + (jnp.arange(pad_e, dtype=jnp.int32) % 16)]
    ).reshape(NS, KT, CHUNK)
    col = jnp.concatenate(
        [edge_index[1], N + (jnp.arange(pad_e, dtype=jnp.int32) % 16)]
    ).reshape(NS, KT, CHUNK)

    x_pad = jnp.concatenate([x, jnp.zeros((NP - N, D), x.dtype)], axis=0)
    zeros2 = jnp.zeros((NP, D), jnp.float32)

    deg2 = _sc_degree(col)
    h1r = _tc_mm(x_pad, W1)
    h1, dinv_col = _tc_scale(h1r, deg2[0].reshape(NP, 1),
                             deg2[1].reshape(NP, 1))
    s1 = _sc_scatter(h1, row, col, zeros2)
    h2 = _tc_layer(s1[0], s1[1], h1, dinv_col, b1.reshape(1, D), W2)
    s2 = _sc_scatter(h2, row, col, zeros2)
    return _tc_final(s2[0], s2[1], h2, dinv_col, b2.reshape(1, D))


# speedup vs baseline: 1.0134x; 1.0084x over previous
import functools

import jax
import jax.numpy as jnp
from jax import lax
from jax.experimental import pallas as pl
from jax.experimental.pallas import tpu as pltpu
from jax.experimental.pallas import tpu_sc as plsc

N = 10000
E = 320000
D = 128

NP = 10240
NC = 2
NS = 16
CHUNK = 128
KT = 160
KA = 80
KW = 80
EP = NS * KT * CHUNK
RPT = NP // NS

BLK = 512
NBLK = NP // BLK

_mesh = plsc.VectorSubcoreMesh(core_axis_name="c", subcore_axis_name="s")


@functools.partial(
    pl.kernel,
    out_type=jax.ShapeDtypeStruct((NC, NP), jnp.float32),
    mesh=_mesh,
    scratch_types=[
        pltpu.VMEM((KW, CHUNK), jnp.int32),
        pltpu.VMEM((CHUNK,), jnp.float32),
        pltpu.VMEM((RPT,), jnp.float32),
        pltpu.VMEM_SHARED((NP,), jnp.float32),
    ],
)
def _sc_degree(col_hbm, deg_hbm, colbuf, ones_v, zero_v, acc):
    c = lax.axis_index("c")
    s = lax.axis_index("s")

    ofs = jnp.where(c == 0, 0, KA)
    cnt = jnp.where(c == 0, KA, KT - KA)
    pltpu.sync_copy(col_hbm.at[s, pl.ds(ofs, KW)], colbuf)

    def fill(j, carry):
        ones_v[pl.ds(j * 16, 16)] = jnp.full((16,), 1.0, jnp.float32)
        return carry
    lax.fori_loop(0, CHUNK // 16, fill, 0)

    def zfill(j, carry):
        zero_v[pl.ds(j * 16, 16)] = jnp.zeros((16,), jnp.float32)
        return carry
    lax.fori_loop(0, RPT // 16, zfill, 0)

    pltpu.sync_copy(zero_v, acc.at[pl.ds(s * RPT, RPT)])
    plsc.subcore_barrier()

    def step(j, carry):
        pltpu.sync_copy(ones_v, acc.at[colbuf.at[j]], add=True)
        return carry
    lax.fori_loop(0, cnt, step, 0)

    plsc.subcore_barrier()
    pltpu.sync_copy(acc.at[pl.ds(s * RPT, RPT)],
                    deg_hbm.at[c, pl.ds(s * RPT, RPT)])


@functools.partial(
    pl.kernel,
    out_type=jax.ShapeDtypeStruct((NC, NP, D), jnp.float32),
    mesh=_mesh,
    scratch_types=[
        pltpu.VMEM((KW, CHUNK), jnp.int32),
        pltpu.VMEM((KW, CHUNK), jnp.int32),
        pltpu.VMEM((CHUNK, D), jnp.float32),
        pltpu.VMEM_SHARED((NP, D), jnp.float32),
        pltpu.SemaphoreType.DMA,
        pltpu.SemaphoreType.DMA,
    ],
)
def _sc_scatter(hp_hbm, row_hbm, col_hbm, z_hbm, out_hbm,
                rowbuf, colbuf, msg0, acc, sem0, semz):
    c = lax.axis_index("c")
    s = lax.axis_index("s")

    ofs = jnp.where(c == 0, 0, KA)
    cnt = jnp.where(c == 0, KA, KT - KA)
    zinit = pltpu.async_copy(z_hbm.at[pl.ds(s * RPT, RPT)],
                             acc.at[pl.ds(s * RPT, RPT)], semz)
    pltpu.sync_copy(row_hbm.at[s, pl.ds(ofs, KW)], rowbuf)
    pltpu.sync_copy(col_hbm.at[s, pl.ds(ofs, KW)], colbuf)
    zinit.wait()
    plsc.subcore_barrier()

    def step(j, carry):
        pltpu.async_copy(hp_hbm.at[rowbuf.at[j]], msg0, sem0).wait()
        pltpu.sync_copy(msg0, acc.at[colbuf.at[j]], add=True)
        return carry
    lax.fori_loop(0, cnt, step, 0)

    plsc.subcore_barrier()
    pltpu.sync_copy(acc.at[pl.ds(s * RPT, RPT)],
                    out_hbm.at[c, pl.ds(s * RPT, RPT)])


def _mm_body(x_ref, w_ref, out_ref):
    out_ref[...] = jnp.dot(x_ref[...], w_ref[...],
                           preferred_element_type=jnp.float32)


def _tc_mm(x, w):
    return pl.pallas_call(
        _mm_body,
        grid=(NBLK,),
        in_specs=[
            pl.BlockSpec((BLK, D), lambda i: (i, 0)),
            pl.BlockSpec((D, D), lambda i: (0, 0)),
        ],
        out_specs=pl.BlockSpec((BLK, D), lambda i: (i, 0)),
        out_shape=jax.ShapeDtypeStruct((NP, D), jnp.float32),
    )(x, w)


def _scale_body(h_ref, d0_ref, d1_ref, hp_ref, dinv_ref):
    dinv = lax.rsqrt(d0_ref[...] + d1_ref[...] + 1.0)
    dinv_ref[...] = dinv
    hp_ref[...] = h_ref[...] * dinv


def _tc_scale(h, d0, d1):
    return pl.pallas_call(
        _scale_body,
        grid=(NBLK,),
        in_specs=[
            pl.BlockSpec((BLK, D), lambda i: (i, 0)),
            pl.BlockSpec((BLK, 1), lambda i: (i, 0)),
            pl.BlockSpec((BLK, 1), lambda i: (i, 0)),
        ],
        out_specs=[
            pl.BlockSpec((BLK, D), lambda i: (i, 0)),
            pl.BlockSpec((BLK, 1), lambda i: (i, 0)),
        ],
        out_shape=[
            jax.ShapeDtypeStruct((NP, D), jnp.float32),
            jax.ShapeDtypeStruct((NP, 1), jnp.float32),
        ],
    )(h, d0, d1)


def _layer_body(s0_ref, s1_ref, hp_ref, dinv_ref, b_ref, w_ref, out_ref):
    a = s0_ref[...] + s1_ref[...] + hp_ref[...]
    a = jnp.maximum(a * dinv_ref[...] + b_ref[...], 0.0)
    h = jnp.dot(a, w_ref[...], preferred_element_type=jnp.float32)
    out_ref[...] = h * dinv_ref[...]


def _tc_layer(s0, s1, hp, dinv_col, b, w):
    return pl.pallas_call(
        _layer_body,
        grid=(NBLK,),
        in_specs=[
            pl.BlockSpec((BLK, D), lambda i: (i, 0)),
            pl.BlockSpec((BLK, D), lambda i: (i, 0)),
            pl.BlockSpec((BLK, D), lambda i: (i, 0)),
            pl.BlockSpec((BLK, 1), lambda i: (i, 0)),
            pl.BlockSpec((1, D), lambda i: (0, 0)),
            pl.BlockSpec((D, D), lambda i: (0, 0)),
        ],
        out_specs=pl.BlockSpec((BLK, D), lambda i: (i, 0)),
        out_shape=jax.ShapeDtypeStruct((NP, D), jnp.float32),
    )(s0, s1, hp, dinv_col, b, w)


def _final_body(s0_ref, s1_ref, hp_ref, dinv_ref, b_ref, out_ref):
    a = s0_ref[...] + s1_ref[...] + hp_ref[...]
    out_ref[...] = jnp.maximum(a * dinv_ref[...] + b_ref[...], 0.0)


def _tc_final(s0, s1, hp, dinv_col, b):
    return pl.pallas_call(
        _final_body,
        grid=(N // 400,),
        in_specs=[
            pl.BlockSpec((400, D), lambda i: (i, 0)),
            pl.BlockSpec((400, D), lambda i: (i, 0)),
            pl.BlockSpec((400, D), lambda i: (i, 0)),
            pl.BlockSpec((400, 1), lambda i: (i, 0)),
            pl.BlockSpec((1, D), lambda i: (0, 0)),
        ],
        out_specs=pl.BlockSpec((400, D), lambda i: (i, 0)),
        out_shape=jax.ShapeDtypeStruct((N, D), jnp.float32),
    )(s0, s1, hp, dinv_col, b)


def kernel(x, edge_index, W1, b1, W2, b2):
    pad_e = EP - E
    row = jnp.concatenate(
        [edge_index[0], N + 16 + (jnp.arange(pad_e, dtype=jnp.int32) % 16)]
    ).reshape(NS, KT, CHUNK)
    col = jnp.concatenate(
        [edge_index[1], N + (jnp.arange(pad_e, dtype=jnp.int32) % 16)]
    ).reshape(NS, KT, CHUNK)

    x_pad = jnp.concatenate([x, jnp.zeros((NP - N, D), x.dtype)], axis=0)
    zeros2 = jnp.zeros((NP, D), jnp.float32)

    deg2 = _sc_degree(col)
    h1r = _tc_mm(x_pad, W1)
    h1, dinv_col = _tc_scale(h1r, deg2[0].reshape(NP, 1),
                             deg2[1].reshape(NP, 1))
    s1 = _sc_scatter(h1, row, col, zeros2)
    h2 = _tc_layer(s1[0], s1[1], h1, dinv_col, b1.reshape(1, D), W2)
    s2 = _sc_scatter(h2, row, col, zeros2)
    return _tc_final(s2[0], s2[1], h2, dinv_col, b2.reshape(1, D))
